# baseline (speedup 1.0000x reference)
import functools
import jax
import jax.numpy as jnp
from jax import lax
from jax.experimental import pallas as pl
from jax.experimental.pallas import tpu as pltpu

B, S, H, I, K = 4, 2048, 768, 3072, 384
TB = 256
NBLK = (B * S) // TB


def _moc_body(x_ref, gw_ref, uw_ref, dw_ref, o_ref, ukey_s, act_s, seed_s):
    i = pl.program_id(0)
    sa = lax.rem(i, 2)
    sb = lax.rem(i + 1, 2)

    def enc(f):
        b = lax.bitcast_convert_type(f, jnp.uint32)
        return jnp.where(b >> 31 == 1, ~b, b | jnp.uint32(0x80000000))

    xb = x_ref[...]
    gate = lax.dot_general(gw_ref[...], xb,
                           (((1,), (1,)), ((), ())),
                           preferred_element_type=jnp.float32)
    up = lax.dot_general(uw_ref[...], xb.astype(jnp.bfloat16),
                         (((1,), (1,)), ((), ())),
                         preferred_element_type=jnp.float32)
    act = (gate * jax.nn.sigmoid(gate) * up).astype(jnp.bfloat16)
    ukey = enc(gate)
    gsub = gate[: I // 4]
    m1 = jnp.sum(gsub, axis=0, keepdims=True) * (4.0 / I)
    m2 = jnp.sum(gsub * gsub, axis=0, keepdims=True) * (4.0 / I)
    sigma = jnp.sqrt(jnp.maximum(m2 - m1 * m1, 1e-30))
    t0 = m1 + 1.1503494 * sigma
    scale = sigma * (1.0 / (I * 0.20594))
    ukey_s[sa] = ukey
    act_s[sa] = act
    seed_s[sa, 0:1, :] = t0
    seed_s[sa, 1:2, :] = scale

    t0b = seed_s[sb, 0:1, :]
    scb = seed_s[sb, 1:2, :]

    def count_ge(cand):
        return jnp.sum((ukey_s[sb] >= cand).astype(jnp.int32),
                       axis=0, keepdims=True)

    lo = jnp.zeros((1, TB), dtype=jnp.uint32)
    hi = jnp.full((1, TB), jnp.uint32(0xFFFFFFFF))
    thr = jnp.zeros((1, TB), dtype=jnp.uint32)
    done = jnp.zeros((1, TB), dtype=jnp.int32)

    def absorb(midu, cnt, lo, hi, thr, done):
        ge = cnt >= K
        nlo = jnp.where(ge, midu, lo)
        nhi = jnp.where(ge, hi, midu)
        closed = nlo + jnp.uint32(1) >= nhi
        hit = cnt == K
        nthr = jnp.where((done == 0) & (hit | closed),
                         jnp.where(hit, midu, nlo), thr)
        ndone = jnp.where(hit | closed, jnp.int32(1), done)
        return nlo, nhi, nthr, ndone

    c0 = count_ge(enc(t0b))
    lo, hi, thr, done = absorb(enc(t0b), c0, lo, hi, thr, done)
    t1 = t0b + (c0 - K).astype(jnp.float32) * scb
    margin = 64.0 * scb
    tb_ = t1 - margin
    cb = count_ge(enc(tb_))
    lo, hi, thr, done = absorb(enc(tb_), cb, lo, hi, thr, done)
    ta_ = t1 + margin
    ca = count_ge(enc(ta_))
    lo, hi, thr, done = absorb(enc(ta_), ca, lo, hi, thr, done)

    for _ in range(12):
        midu = lo + ((hi - lo) >> 1)
        cnt = count_ge(midu)
        lo, hi, thr, done = absorb(midu, cnt, lo, hi, thr, done)

    thr = jnp.where(done == 1, thr, lo)

    mask = ukey_s[sb] >= thr
    masked = jnp.where(mask, act_s[sb][...], jnp.bfloat16(0.0))
    o_ref[...] = lax.dot_general(masked, dw_ref[...],
                                 (((0,), (1,)), ((), ())),
                                 preferred_element_type=jnp.float32)


@jax.jit
def kernel(x, gate_w, up_w, down_w):
    b, s, h = x.shape
    T = b * s
    x2 = x.reshape(T, h)
    up_w = up_w.astype(jnp.bfloat16)
    down_w = down_w.astype(jnp.bfloat16)
    out = pl.pallas_call(
        _moc_body,
        grid=(NBLK + 1,),
        in_specs=[
            pl.BlockSpec((TB, H), lambda i: (jnp.minimum(i, NBLK - 1), 0)),
            pl.BlockSpec((I, H), lambda i: (0, 0)),
            pl.BlockSpec((I, H), lambda i: (0, 0)),
            pl.BlockSpec((H, I), lambda i: (0, 0)),
        ],
        out_specs=pl.BlockSpec(
            (TB, H),
            lambda i: (jnp.clip(i - 1, 0, NBLK - 1), 0)),
        out_shape=jax.ShapeDtypeStruct((T, H), jnp.float32),
        scratch_shapes=[
            pltpu.VMEM((2, I, TB), jnp.uint32),
            pltpu.VMEM((2, I, TB), jnp.bfloat16),
            pltpu.VMEM((2, 8, TB), jnp.float32),
        ],
    )(x2, gate_w, up_w, down_w)
    return out.reshape(b, s, h)


# speedup vs baseline: 1.1586x; 1.1586x over previous
import functools
import jax
import jax.numpy as jnp
from jax import lax
from jax.experimental import pallas as pl
from jax.experimental.pallas import tpu as pltpu

B, S, H, I, K = 4, 2048, 768, 3072, 384
TB = 512
NBLK = (B * S) // TB


def _moc_body(x_ref, gw_ref, uw_ref, dw_ref, o_ref, ukey_s, act_s, seed_s):
    i = pl.program_id(0)
    sa = lax.rem(i, 2)
    sb = lax.rem(i + 1, 2)

    def enc(f):
        b = lax.bitcast_convert_type(f, jnp.uint32)
        return jnp.where(b >> 31 == 1, ~b, b | jnp.uint32(0x80000000))

    xb = x_ref[...]
    gate = lax.dot_general(gw_ref[...], xb,
                           (((1,), (1,)), ((), ())),
                           preferred_element_type=jnp.float32)
    up = lax.dot_general(uw_ref[...], xb.astype(jnp.bfloat16),
                         (((1,), (1,)), ((), ())),
                         preferred_element_type=jnp.float32)
    act = (gate * jax.nn.sigmoid(gate) * up).astype(jnp.bfloat16)
    ukey = enc(gate)
    m1 = jnp.sum(gate, axis=0, keepdims=True) * (1.0 / I)
    m2 = jnp.sum(gate * gate, axis=0, keepdims=True) * (1.0 / I)
    sigma = jnp.sqrt(jnp.maximum(m2 - m1 * m1, 1e-30))
    t0 = m1 + 1.1503494 * sigma
    scale = sigma * (1.0 / (I * 0.20594))
    ukey_s[sa] = ukey
    act_s[sa] = act
    seed_s[sa, 0:1, :] = t0
    seed_s[sa, 1:2, :] = scale

    t0b = seed_s[sb, 0:1, :]
    scb = seed_s[sb, 1:2, :]

    def count_ge(cand):
        return jnp.sum((ukey_s[sb] >= cand).astype(jnp.int32),
                       axis=0, keepdims=True)

    lo = jnp.zeros((1, TB), dtype=jnp.uint32)
    hi = jnp.full((1, TB), jnp.uint32(0xFFFFFFFF))
    thr = jnp.zeros((1, TB), dtype=jnp.uint32)
    done = jnp.zeros((1, TB), dtype=jnp.int32)

    def absorb(midu, cnt, lo, hi, thr, done):
        ge = cnt >= K
        nlo = jnp.where(ge, midu, lo)
        nhi = jnp.where(ge, hi, midu)
        closed = nlo + jnp.uint32(1) >= nhi
        hit = cnt == K
        nthr = jnp.where((done == 0) & (hit | closed),
                         jnp.where(hit, midu, nlo), thr)
        ndone = jnp.where(hit | closed, jnp.int32(1), done)
        return nlo, nhi, nthr, ndone

    c0 = count_ge(enc(t0b))
    lo, hi, thr, done = absorb(enc(t0b), c0, lo, hi, thr, done)
    t1 = t0b + (c0 - K).astype(jnp.float32) * scb
    margin = 64.0 * scb
    tb_ = t1 - margin
    cb = count_ge(enc(tb_))
    lo, hi, thr, done = absorb(enc(tb_), cb, lo, hi, thr, done)
    ta_ = t1 + margin
    ca = count_ge(enc(ta_))
    lo, hi, thr, done = absorb(enc(ta_), ca, lo, hi, thr, done)

    for _ in range(14):
        midu = lo + ((hi - lo) >> 1)
        cnt = count_ge(midu)
        lo, hi, thr, done = absorb(midu, cnt, lo, hi, thr, done)

    thr = jnp.where(done == 1, thr, lo)

    mask = ukey_s[sb] >= thr
    masked = jnp.where(mask, act_s[sb][...], jnp.bfloat16(0.0))
    o_ref[...] = lax.dot_general(masked, dw_ref[...],
                                 (((0,), (1,)), ((), ())),
                                 preferred_element_type=jnp.float32)


@jax.jit
def kernel(x, gate_w, up_w, down_w):
    b, s, h = x.shape
    T = b * s
    x2 = x.reshape(T, h)
    up_w = up_w.astype(jnp.bfloat16)
    down_w = down_w.astype(jnp.bfloat16)
    out = pl.pallas_call(
        _moc_body,
        grid=(NBLK + 1,),
        in_specs=[
            pl.BlockSpec((TB, H), lambda i: (jnp.minimum(i, NBLK - 1), 0)),
            pl.BlockSpec((I, H), lambda i: (0, 0)),
            pl.BlockSpec((I, H), lambda i: (0, 0)),
            pl.BlockSpec((H, I), lambda i: (0, 0)),
        ],
        out_specs=pl.BlockSpec(
            (TB, H),
            lambda i: (jnp.clip(i - 1, 0, NBLK - 1), 0)),
        out_shape=jax.ShapeDtypeStruct((T, H), jnp.float32),
        scratch_shapes=[
            pltpu.VMEM((2, I, TB), jnp.uint32),
            pltpu.VMEM((2, I, TB), jnp.bfloat16),
            pltpu.VMEM((2, 8, TB), jnp.float32),
        ],
    )(x2, gate_w, up_w, down_w)
    return out.reshape(b, s, h)
